# baseline (device time: 114053 ns/iter reference)
import jax
import jax.numpy as jnp
from jax import lax
from jax.experimental import pallas as pl
from jax.experimental.pallas import tpu as pltpu

N_DEV = 8


def kernel(A, B):
    m, k = A.shape
    _, n = B.shape
    ch = m // N_DEV

    def body(a_ref, b_ref, out_ref, part_ref, rs_buf,
             rs_send_sems, rs_recv_sems, ag_send_sems, ag_recv_sems):
        my = lax.axis_index("i")
        left = lax.rem(my + N_DEV - 1, N_DEV)
        right = lax.rem(my + 1, N_DEV)

        barrier_sem = pltpu.get_barrier_semaphore()
        for nbr in (left, right):
            pl.semaphore_signal(
                barrier_sem, inc=1,
                device_id=(nbr,), device_id_type=pl.DeviceIdType.MESH,
            )
        pl.semaphore_wait(barrier_sem, 2)

        part_ref[...] = jnp.dot(
            a_ref[...], b_ref[...], preferred_element_type=jnp.float32
        )

        def part_chunk(c):
            return part_ref.at[pl.ds(c * ch, ch), :]

        for s in range(N_DEV - 1):
            src = part_chunk(my) if s == 0 else rs_buf.at[s - 1]
            rdma = pltpu.make_async_remote_copy(
                src_ref=src,
                dst_ref=rs_buf.at[s],
                send_sem=rs_send_sems.at[s],
                recv_sem=rs_recv_sems.at[s],
                device_id=(right,),
                device_id_type=pl.DeviceIdType.MESH,
            )
            rdma.start()
            rdma.wait()
            c_recv = lax.rem(my + N_DEV - 1 - s, N_DEV)
            rs_buf[s, :, :] = rs_buf[s, :, :] + part_ref[pl.ds(c_recv * ch, ch), :]

        c_own = lax.rem(my + 1, N_DEV)
        z = rs_buf[N_DEV - 2, :, :]
        out_ref[pl.ds(c_own * ch, ch), :] = z * jax.nn.sigmoid(z)

        for h in range(N_DEV - 1):
            c_fwd = lax.rem(my + 1 - h + N_DEV, N_DEV)
            rows = pl.ds(c_fwd * ch, ch)
            rdma = pltpu.make_async_remote_copy(
                src_ref=out_ref.at[rows, :],
                dst_ref=out_ref.at[rows, :],
                send_sem=ag_send_sems.at[h],
                recv_sem=ag_recv_sems.at[h],
                device_id=(right,),
                device_id_type=pl.DeviceIdType.MESH,
            )
            rdma.start()
            rdma.wait()

    return pl.pallas_call(
        body,
        out_shape=jax.ShapeDtypeStruct((m, n), jnp.float32),
        in_specs=[
            pl.BlockSpec(memory_space=pltpu.VMEM),
            pl.BlockSpec(memory_space=pltpu.VMEM),
        ],
        out_specs=pl.BlockSpec(memory_space=pltpu.VMEM),
        scratch_shapes=[
            pltpu.VMEM((m, n), jnp.float32),
            pltpu.VMEM((N_DEV - 1, ch, n), jnp.float32),
            pltpu.SemaphoreType.DMA((N_DEV - 1,)),
            pltpu.SemaphoreType.DMA((N_DEV - 1,)),
            pltpu.SemaphoreType.DMA((N_DEV - 1,)),
            pltpu.SemaphoreType.DMA((N_DEV - 1,)),
        ],
        compiler_params=pltpu.CompilerParams(collective_id=0),
    )(A, B)


# device time: 51480 ns/iter; 2.2155x vs baseline; 2.2155x over previous
import jax
import jax.numpy as jnp
from jax import lax
from jax.experimental import pallas as pl
from jax.experimental.pallas import tpu as pltpu

N_DEV = 8
CH = 128

X, Y, Z = 1, 3, 4
ORDERS = ((X, Y, Z), (Y, Z, X), (Z, X, Y))
STRIPES = ((0, 384), (384, 384), (768, 256))


def _span(masks):
    out = [0]
    for m in masks:
        out = out + [v ^ m for v in out]
    return out


def kernel(A, B):
    m, _ = A.shape
    _, n = B.shape

    def body(a_ref, b_ref, out_ref, part_ref, buf0, buf1, buf2,
             rs_send, rs_recv, ag_send, ag_recv):
        bufs = (buf0, buf1, buf2)
        my = lax.axis_index("i")

        barrier_sem = pltpu.get_barrier_semaphore()
        for mask in (X, Y, Z):
            pl.semaphore_signal(
                barrier_sem, inc=1,
                device_id=(my ^ mask,), device_id_type=pl.DeviceIdType.MESH,
            )
        pl.semaphore_wait(barrier_sem, 3)

        part_ref[...] = jnp.dot(
            a_ref[...], b_ref[...], preferred_element_type=jnp.float32
        )

        for t in range(3):
            rdmas = []
            for r, order in enumerate(ORDERS):
                mask = order[t]
                rest = _span(order[t + 1:])
                c0, w = STRIPES[r]
                cols = pl.ds(c0, w)
                slot0 = 8 - 2 * len(rest)
                for k, s in enumerate(rest):
                    c_send = my ^ mask ^ s
                    rdma = pltpu.make_async_remote_copy(
                        src_ref=part_ref.at[pl.ds(c_send * CH, CH), cols],
                        dst_ref=bufs[r].at[slot0 + k],
                        send_sem=rs_send.at[r, slot0 + k],
                        recv_sem=rs_recv.at[r, slot0 + k],
                        device_id=(my ^ mask,),
                        device_id_type=pl.DeviceIdType.MESH,
                    )
                    rdma.start()
                    rdmas.append(rdma)
            for rdma in rdmas:
                rdma.wait()
            for r, order in enumerate(ORDERS):
                rest = _span(order[t + 1:])
                c0, w = STRIPES[r]
                cols = pl.ds(c0, w)
                slot0 = 8 - 2 * len(rest)
                for k, s in enumerate(rest):
                    c_keep = my ^ s
                    rows = pl.ds(c_keep * CH, CH)
                    part_ref[rows, cols] = (
                        part_ref[rows, cols] + bufs[r][slot0 + k, :, :w]
                    )

        rows = pl.ds(my * CH, CH)
        z = part_ref[rows, :]
        out_ref[rows, :] = z * jax.nn.sigmoid(z)

        for t in range(3):
            rdmas = []
            for r, order in enumerate(ORDERS):
                ag_order = order[::-1]
                mask = ag_order[t]
                have = _span(ag_order[:t])
                c0, w = STRIPES[r]
                cols = pl.ds(c0, w)
                slot0 = len(have) - 1
                for k, s in enumerate(have):
                    c_send = my ^ s
                    crows = pl.ds(c_send * CH, CH)
                    rdma = pltpu.make_async_remote_copy(
                        src_ref=out_ref.at[crows, cols],
                        dst_ref=out_ref.at[crows, cols],
                        send_sem=ag_send.at[r, slot0 + k],
                        recv_sem=ag_recv.at[r, slot0 + k],
                        device_id=(my ^ mask,),
                        device_id_type=pl.DeviceIdType.MESH,
                    )
                    rdma.start()
                    rdmas.append(rdma)
            for rdma in rdmas:
                rdma.wait()

    return pl.pallas_call(
        body,
        out_shape=jax.ShapeDtypeStruct((m, n), jnp.float32),
        in_specs=[
            pl.BlockSpec(memory_space=pltpu.VMEM),
            pl.BlockSpec(memory_space=pltpu.VMEM),
        ],
        out_specs=pl.BlockSpec(memory_space=pltpu.VMEM),
        scratch_shapes=[
            pltpu.VMEM((m, n), jnp.float32),
            pltpu.VMEM((7, CH, 384), jnp.float32),
            pltpu.VMEM((7, CH, 384), jnp.float32),
            pltpu.VMEM((7, CH, 256), jnp.float32),
            pltpu.SemaphoreType.DMA((3, 7)),
            pltpu.SemaphoreType.DMA((3, 7)),
            pltpu.SemaphoreType.DMA((3, 7)),
            pltpu.SemaphoreType.DMA((3, 7)),
        ],
        compiler_params=pltpu.CompilerParams(collective_id=0),
    )(A, B)


# device time: 50613 ns/iter; 2.2534x vs baseline; 1.0171x over previous
import jax
import jax.numpy as jnp
from jax import lax
from jax.experimental import pallas as pl
from jax.experimental.pallas import tpu as pltpu

N_DEV = 8
CH = 128

X, Y, Z = 1, 3, 4
ORDERS = ((X, Y, Z), (Y, Z, X), (Z, X, Y))
STRIPES = ((0, 384), (384, 384), (768, 256))


def _span(masks):
    out = [0]
    for m in masks:
        out = out + [v ^ m for v in out]
    return out


def kernel(A, B):
    m, _ = A.shape
    _, n = B.shape

    def body(a_ref, b_ref, out_ref, part_ref, buf0, buf1, buf2,
             rs_send, rs_recv, ag_send, ag_recv):
        bufs = (buf0, buf1, buf2)
        my = lax.axis_index("i")

        barrier_sem = pltpu.get_barrier_semaphore()
        for mask in (X, Y, Z):
            pl.semaphore_signal(
                barrier_sem, inc=1,
                device_id=(my ^ mask,), device_id_type=pl.DeviceIdType.MESH,
            )
        pl.semaphore_wait(barrier_sem, 3)

        def issue_rs(r, t):
            order = ORDERS[r]
            mask = order[t]
            rest = _span(order[t + 1:])
            c0, w = STRIPES[r]
            slot0 = 8 - 2 * len(rest)
            out = []
            for k, s in enumerate(rest):
                c_send = my ^ mask ^ s
                rdma = pltpu.make_async_remote_copy(
                    src_ref=part_ref.at[pl.ds(c_send * CH, CH), pl.ds(c0, w)],
                    dst_ref=bufs[r].at[slot0 + k],
                    send_sem=rs_send.at[r, slot0 + k],
                    recv_sem=rs_recv.at[r, slot0 + k],
                    device_id=(my ^ mask,),
                    device_id_type=pl.DeviceIdType.MESH,
                )
                rdma.start()
                out.append((rdma, slot0 + k, s))
            return out

        def issue_ag(r, t):
            order = ORDERS[r]
            ag_order = order[::-1]
            mask = ag_order[t]
            have = _span(ag_order[:t])
            c0, w = STRIPES[r]
            slot0 = len(have) - 1
            out = []
            for k, s in enumerate(have):
                crows = pl.ds((my ^ s) * CH, CH)
                rdma = pltpu.make_async_remote_copy(
                    src_ref=out_ref.at[crows, pl.ds(c0, w)],
                    dst_ref=out_ref.at[crows, pl.ds(c0, w)],
                    send_sem=ag_send.at[r, slot0 + k],
                    recv_sem=ag_recv.at[r, slot0 + k],
                    device_id=(my ^ mask,),
                    device_id_type=pl.DeviceIdType.MESH,
                )
                rdma.start()
                out.append(rdma)
            return out

        pending = {}
        for r in range(3):
            c0, w = STRIPES[r]
            part_ref[:, pl.ds(c0, w)] = jnp.dot(
                a_ref[...], b_ref[:, pl.ds(c0, w)],
                preferred_element_type=jnp.float32,
            )
            pending[r] = issue_rs(r, 0)

        for t in range(3):
            for r in range(3):
                c0, w = STRIPES[r]
                for rdma, slot, s in pending[r]:
                    rdma.wait()
                    rows = pl.ds((my ^ s) * CH, CH)
                    part_ref[rows, pl.ds(c0, w)] = (
                        part_ref[rows, pl.ds(c0, w)] + bufs[r][slot, :, :]
                    )
                if t < 2:
                    pending[r] = issue_rs(r, t + 1)
                else:
                    rows = pl.ds(my * CH, CH)
                    z = part_ref[rows, pl.ds(c0, w)]
                    out_ref[rows, pl.ds(c0, w)] = z * jax.nn.sigmoid(z)
                    pending[r] = issue_ag(r, 0)

        for t in range(3):
            for r in range(3):
                for rdma in pending[r]:
                    rdma.wait()
                if t < 2:
                    pending[r] = issue_ag(r, t + 1)

    return pl.pallas_call(
        body,
        out_shape=jax.ShapeDtypeStruct((m, n), jnp.float32),
        in_specs=[
            pl.BlockSpec(memory_space=pltpu.VMEM),
            pl.BlockSpec(memory_space=pltpu.VMEM),
        ],
        out_specs=pl.BlockSpec(memory_space=pltpu.VMEM),
        scratch_shapes=[
            pltpu.VMEM((m, n), jnp.float32),
            pltpu.VMEM((7, CH, 384), jnp.float32),
            pltpu.VMEM((7, CH, 384), jnp.float32),
            pltpu.VMEM((7, CH, 256), jnp.float32),
            pltpu.SemaphoreType.DMA((3, 7)),
            pltpu.SemaphoreType.DMA((3, 7)),
            pltpu.SemaphoreType.DMA((3, 7)),
            pltpu.SemaphoreType.DMA((3, 7)),
        ],
        compiler_params=pltpu.CompilerParams(collective_id=0),
    )(A, B)


# device time: 36130 ns/iter; 3.1567x vs baseline; 1.4009x over previous
import jax
import jax.numpy as jnp
from jax import lax
from jax.experimental import pallas as pl
from jax.experimental.pallas import tpu as pltpu

N_DEV = 8
CH = 128

X, Y, Z = 1, 3, 4
ORDERS = ((X, Y, Z), (Y, Z, X), (Z, X, Y))
STRIPES = ((0, 384), (384, 384), (768, 256))


def _span(masks):
    out = [0]
    for m in masks:
        out = out + [v ^ m for v in out]
    return out


def kernel(A, B):
    m, _ = A.shape
    _, n = B.shape

    def body(a_ref, b_ref, out_ref, part_ref,
             rbuf0, rbuf1, rbuf2, sbuf0, sbuf1, sbuf2,
             gbuf0, gbuf1, gbuf2,
             rs_send, rs_recv, ag_send, ag_recv):
        rbufs = (rbuf0, rbuf1, rbuf2)
        sbufs = (sbuf0, sbuf1, sbuf2)
        gbufs = (gbuf0, gbuf1, gbuf2)
        my = lax.axis_index("i")

        barrier_sem = pltpu.get_barrier_semaphore()
        for mask in (X, Y, Z):
            pl.semaphore_signal(
                barrier_sem, inc=1,
                device_id=(my ^ mask,), device_id_type=pl.DeviceIdType.MESH,
            )
        pl.semaphore_wait(barrier_sem, 3)

        def issue_rs(r, t):
            order = ORDERS[r]
            mask = order[t]
            rest = _span(order[t + 1:])
            c0, w = STRIPES[r]
            slot0 = 8 - 2 * len(rest)
            out = []
            for k, s in enumerate(rest):
                slot = slot0 + k
                c_send = my ^ mask ^ s
                sbufs[r][slot, :, :] = part_ref[
                    pl.ds(c_send * CH, CH), pl.ds(c0, w)
                ].astype(jnp.bfloat16)
                rdma = pltpu.make_async_remote_copy(
                    src_ref=sbufs[r].at[slot],
                    dst_ref=rbufs[r].at[slot],
                    send_sem=rs_send.at[r, slot],
                    recv_sem=rs_recv.at[r, slot],
                    device_id=(my ^ mask,),
                    device_id_type=pl.DeviceIdType.MESH,
                )
                rdma.start()
                out.append((rdma, slot, s))
            return out

        def issue_ag(r, t):
            order = ORDERS[r]
            ag_order = order[::-1]
            mask = ag_order[t]
            have = _span(ag_order[:t])
            c0, w = STRIPES[r]
            slot0 = len(have) - 1
            out = []
            for k, s in enumerate(have):
                c_send = my ^ s
                rdma = pltpu.make_async_remote_copy(
                    src_ref=gbufs[r].at[c_send],
                    dst_ref=gbufs[r].at[c_send],
                    send_sem=ag_send.at[r, slot0 + k],
                    recv_sem=ag_recv.at[r, slot0 + k],
                    device_id=(my ^ mask,),
                    device_id_type=pl.DeviceIdType.MESH,
                )
                rdma.start()
                out.append((rdma, my ^ mask ^ s))
            return out

        pending = {}
        for r in range(3):
            c0, w = STRIPES[r]
            part_ref[:, pl.ds(c0, w)] = jnp.dot(
                a_ref[...], b_ref[:, pl.ds(c0, w)],
                preferred_element_type=jnp.float32,
            )
            pending[r] = issue_rs(r, 0)

        for t in range(3):
            for r in range(3):
                c0, w = STRIPES[r]
                for rdma, slot, s in pending[r]:
                    rdma.wait()
                    rows = pl.ds((my ^ s) * CH, CH)
                    part_ref[rows, pl.ds(c0, w)] = (
                        part_ref[rows, pl.ds(c0, w)]
                        + rbufs[r][slot, :, :].astype(jnp.float32)
                    )
                if t < 2:
                    pending[r] = issue_rs(r, t + 1)
                else:
                    rows = pl.ds(my * CH, CH)
                    z = part_ref[rows, pl.ds(c0, w)]
                    act = z * jax.nn.sigmoid(z)
                    out_ref[rows, pl.ds(c0, w)] = act
                    gbufs[r][pl.ds(my, 1), :, :] = act.astype(jnp.bfloat16)[
                        None, :, :
                    ]
                    pending[r] = issue_ag(r, 0)

        for t in range(3):
            for r in range(3):
                c0, w = STRIPES[r]
                done = []
                for rdma, c in pending[r]:
                    rdma.wait()
                    done.append(c)
                for c in done:
                    out_ref[pl.ds(c * CH, CH), pl.ds(c0, w)] = gbufs[r][
                        c, :, :
                    ].astype(jnp.float32)
                if t < 2:
                    pending[r] = issue_ag(r, t + 1)

    return pl.pallas_call(
        body,
        out_shape=jax.ShapeDtypeStruct((m, n), jnp.float32),
        in_specs=[
            pl.BlockSpec(memory_space=pltpu.VMEM),
            pl.BlockSpec(memory_space=pltpu.VMEM),
        ],
        out_specs=pl.BlockSpec(memory_space=pltpu.VMEM),
        scratch_shapes=[
            pltpu.VMEM((m, n), jnp.float32),
            pltpu.VMEM((7, CH, 384), jnp.bfloat16),
            pltpu.VMEM((7, CH, 384), jnp.bfloat16),
            pltpu.VMEM((7, CH, 256), jnp.bfloat16),
            pltpu.VMEM((7, CH, 384), jnp.bfloat16),
            pltpu.VMEM((7, CH, 384), jnp.bfloat16),
            pltpu.VMEM((7, CH, 256), jnp.bfloat16),
            pltpu.VMEM((N_DEV, CH, 384), jnp.bfloat16),
            pltpu.VMEM((N_DEV, CH, 384), jnp.bfloat16),
            pltpu.VMEM((N_DEV, CH, 256), jnp.bfloat16),
            pltpu.SemaphoreType.DMA((3, 7)),
            pltpu.SemaphoreType.DMA((3, 7)),
            pltpu.SemaphoreType.DMA((3, 7)),
            pltpu.SemaphoreType.DMA((3, 7)),
        ],
        compiler_params=pltpu.CompilerParams(collective_id=0),
    )(A, B)


# device time: 36086 ns/iter; 3.1606x vs baseline; 1.0012x over previous
import jax
import jax.numpy as jnp
from jax import lax
from jax.experimental import pallas as pl
from jax.experimental.pallas import tpu as pltpu

N_DEV = 8
CH = 128

X, Y, Z = 1, 3, 4
ORDERS = ((X, Y, Z), (Y, Z, X), (Z, X, Y))
STRIPES = ((0, 384), (384, 384), (768, 256))


def _span(masks):
    out = [0]
    for m in masks:
        out = out + [v ^ m for v in out]
    return out


def kernel(A, B):
    m, _ = A.shape
    _, n = B.shape

    def body(a_ref, b_ref, out_ref, part_ref,
             rbuf0, rbuf1, rbuf2, sbuf0, sbuf1, sbuf2,
             gbuf0, gbuf1, gbuf2,
             rs_send, rs_recv, ag_send, ag_recv):
        rbufs = (rbuf0, rbuf1, rbuf2)
        sbufs = (sbuf0, sbuf1, sbuf2)
        gbufs = (gbuf0, gbuf1, gbuf2)
        my = lax.axis_index("i")
        all_rdmas = []

        barrier_sem = pltpu.get_barrier_semaphore()
        for mask in (X, Y, Z):
            pl.semaphore_signal(
                barrier_sem, inc=1,
                device_id=(my ^ mask,), device_id_type=pl.DeviceIdType.MESH,
            )
        pl.semaphore_wait(barrier_sem, 3)

        def issue_rs(r, t, prev_slot_by_s):
            order = ORDERS[r]
            mask = order[t]
            rest = _span(order[t + 1:])
            c0, w = STRIPES[r]
            slot0 = 8 - 2 * len(rest)
            out = []
            for k, s in enumerate(rest):
                slot = slot0 + k
                c_send = my ^ mask ^ s
                rows = pl.ds(c_send * CH, CH)
                val = part_ref[rows, pl.ds(c0, w)]
                if t > 0:
                    pslot = prev_slot_by_s[mask ^ s]
                    val = val + rbufs[r][pslot, :, :].astype(jnp.float32)
                sbufs[r][slot, :, :] = val.astype(jnp.bfloat16)
                rdma = pltpu.make_async_remote_copy(
                    src_ref=sbufs[r].at[slot],
                    dst_ref=rbufs[r].at[slot],
                    send_sem=rs_send.at[r, slot],
                    recv_sem=rs_recv.at[r, slot],
                    device_id=(my ^ mask,),
                    device_id_type=pl.DeviceIdType.MESH,
                )
                rdma.start()
                all_rdmas.append(rdma)
                out.append((rdma, slot, s))
            return out

        def issue_ag(r, t):
            order = ORDERS[r]
            ag_order = order[::-1]
            mask = ag_order[t]
            have = _span(ag_order[:t])
            c0, w = STRIPES[r]
            slot0 = len(have) - 1
            out = []
            for k, s in enumerate(have):
                c_send = my ^ s
                rdma = pltpu.make_async_remote_copy(
                    src_ref=gbufs[r].at[c_send],
                    dst_ref=gbufs[r].at[c_send],
                    send_sem=ag_send.at[r, slot0 + k],
                    recv_sem=ag_recv.at[r, slot0 + k],
                    device_id=(my ^ mask,),
                    device_id_type=pl.DeviceIdType.MESH,
                )
                rdma.start()
                all_rdmas.append(rdma)
                out.append((rdma, my ^ mask ^ s))
            return out

        pending = {}
        for r in range(3):
            c0, w = STRIPES[r]
            part_ref[:, pl.ds(c0, w)] = jnp.dot(
                a_ref[...], b_ref[:, pl.ds(c0, w)],
                preferred_element_type=jnp.float32,
            )
            pending[r] = issue_rs(r, 0, None)

        for t in range(3):
            for r in range(3):
                order = ORDERS[r]
                c0, w = STRIPES[r]
                cur = pending[r]
                slot_by_s = {s: slot for (_, slot, s) in cur}
                if t < 2:
                    fwd = {order[t + 1] ^ s2 for s2 in _span(order[t + 2:])}
                    for rdma, slot, s in cur:
                        if s in fwd:
                            rdma.wait_recv()
                    pending[r] = issue_rs(r, t + 1, slot_by_s)
                    for rdma, slot, s in cur:
                        if s not in fwd:
                            rdma.wait_recv()
                            rows = pl.ds((my ^ s) * CH, CH)
                            part_ref[rows, pl.ds(c0, w)] = (
                                part_ref[rows, pl.ds(c0, w)]
                                + rbufs[r][slot, :, :].astype(jnp.float32)
                            )
                else:
                    rdma, slot, s = cur[0]
                    rdma.wait_recv()
                    rows = pl.ds(my * CH, CH)
                    z = (
                        part_ref[rows, pl.ds(c0, w)]
                        + rbufs[r][slot, :, :].astype(jnp.float32)
                    )
                    act = z * jax.nn.sigmoid(z)
                    out_ref[rows, pl.ds(c0, w)] = act
                    gbufs[r][pl.ds(my, 1), :, :] = act.astype(jnp.bfloat16)[
                        None, :, :
                    ]
                    pending[r] = issue_ag(r, 0)

        for t in range(3):
            for r in range(3):
                c0, w = STRIPES[r]
                arrived = []
                for rdma, c in pending[r]:
                    rdma.wait_recv()
                    arrived.append(c)
                pending[r] = issue_ag(r, t + 1) if t < 2 else []
                for c in arrived:
                    out_ref[pl.ds(c * CH, CH), pl.ds(c0, w)] = gbufs[r][
                        c, :, :
                    ].astype(jnp.float32)

        for rdma in all_rdmas:
            rdma.wait_send()

    return pl.pallas_call(
        body,
        out_shape=jax.ShapeDtypeStruct((m, n), jnp.float32),
        in_specs=[
            pl.BlockSpec(memory_space=pltpu.VMEM),
            pl.BlockSpec(memory_space=pltpu.VMEM),
        ],
        out_specs=pl.BlockSpec(memory_space=pltpu.VMEM),
        scratch_shapes=[
            pltpu.VMEM((m, n), jnp.float32),
            pltpu.VMEM((7, CH, 384), jnp.bfloat16),
            pltpu.VMEM((7, CH, 384), jnp.bfloat16),
            pltpu.VMEM((7, CH, 256), jnp.bfloat16),
            pltpu.VMEM((7, CH, 384), jnp.bfloat16),
            pltpu.VMEM((7, CH, 384), jnp.bfloat16),
            pltpu.VMEM((7, CH, 256), jnp.bfloat16),
            pltpu.VMEM((N_DEV, CH, 384), jnp.bfloat16),
            pltpu.VMEM((N_DEV, CH, 384), jnp.bfloat16),
            pltpu.VMEM((N_DEV, CH, 256), jnp.bfloat16),
            pltpu.SemaphoreType.DMA((3, 7)),
            pltpu.SemaphoreType.DMA((3, 7)),
            pltpu.SemaphoreType.DMA((3, 7)),
            pltpu.SemaphoreType.DMA((3, 7)),
        ],
        compiler_params=pltpu.CompilerParams(collective_id=0),
    )(A, B)


# device time: 35727 ns/iter; 3.1923x vs baseline; 1.0100x over previous
import jax
import jax.numpy as jnp
from jax import lax
from jax.experimental import pallas as pl
from jax.experimental.pallas import tpu as pltpu

N_DEV = 8
CH = 128

X, Y, Z = 1, 3, 4
ORDERS = ((X, Y, Z), (Y, Z, X), (Z, X, Y))
STRIPES = ((0, 384), (384, 384), (768, 256))


def _span(masks):
    out = [0]
    for m in masks:
        out = out + [v ^ m for v in out]
    return out


def kernel(A, B):
    m, _ = A.shape
    _, n = B.shape

    def body(a_ref, b_ref, out_ref, part_ref,
             rbuf0, rbuf1, rbuf2, sbuf0, sbuf1, sbuf2,
             gbuf0, gbuf1, gbuf2,
             rs_send, rs_recv, ag_send, ag_recv):
        rbufs = (rbuf0, rbuf1, rbuf2)
        sbufs = (sbuf0, sbuf1, sbuf2)
        gbufs = (gbuf0, gbuf1, gbuf2)
        my = lax.axis_index("i")
        all_rdmas = []

        barrier_sem = pltpu.get_barrier_semaphore()
        for mask in (X, Y, Z):
            pl.semaphore_signal(
                barrier_sem, inc=1,
                device_id=(my ^ mask,), device_id_type=pl.DeviceIdType.MESH,
            )

        def issue_rs(r, t, prev_slot_by_s, prestaged=False):
            order = ORDERS[r]
            mask = order[t]
            rest = _span(order[t + 1:])
            c0, w = STRIPES[r]
            slot0 = 8 - 2 * len(rest)
            out = []
            for k, s in enumerate(rest):
                slot = slot0 + k
                c_send = my ^ mask ^ s
                if not prestaged:
                    rows = pl.ds(c_send * CH, CH)
                    val = part_ref[rows, pl.ds(c0, w)]
                    if t > 0:
                        pslot = prev_slot_by_s[mask ^ s]
                        val = val + rbufs[r][pslot, :, :].astype(jnp.float32)
                    sbufs[r][slot, :, :] = val.astype(jnp.bfloat16)
                rdma = pltpu.make_async_remote_copy(
                    src_ref=sbufs[r].at[slot],
                    dst_ref=rbufs[r].at[slot],
                    send_sem=rs_send.at[r, slot],
                    recv_sem=rs_recv.at[r, slot],
                    device_id=(my ^ mask,),
                    device_id_type=pl.DeviceIdType.MESH,
                )
                rdma.start()
                all_rdmas.append(rdma)
                out.append((rdma, slot, s))
            return out

        def issue_ag(r, t):
            order = ORDERS[r]
            ag_order = order[::-1]
            mask = ag_order[t]
            have = _span(ag_order[:t])
            c0, w = STRIPES[r]
            slot0 = len(have) - 1
            out = []
            for k, s in enumerate(have):
                c_send = my ^ s
                rdma = pltpu.make_async_remote_copy(
                    src_ref=gbufs[r].at[c_send],
                    dst_ref=gbufs[r].at[c_send],
                    send_sem=ag_send.at[r, slot0 + k],
                    recv_sem=ag_recv.at[r, slot0 + k],
                    device_id=(my ^ mask,),
                    device_id_type=pl.DeviceIdType.MESH,
                )
                rdma.start()
                all_rdmas.append(rdma)
                out.append((rdma, my ^ mask ^ s))
            return out

        for r in range(3):
            order = ORDERS[r]
            c0, w = STRIPES[r]
            part_ref[:, pl.ds(c0, w)] = jnp.dot(
                a_ref[...], b_ref[:, pl.ds(c0, w)],
                preferred_element_type=jnp.float32,
            )
            for k, s in enumerate(_span(order[1:])):
                c_send = my ^ order[0] ^ s
                sbufs[r][k, :, :] = part_ref[
                    pl.ds(c_send * CH, CH), pl.ds(c0, w)
                ].astype(jnp.bfloat16)
        pl.semaphore_wait(barrier_sem, 3)
        pending = {r: issue_rs(r, 0, None, prestaged=True) for r in range(3)}

        for t in range(3):
            for r in range(3):
                order = ORDERS[r]
                c0, w = STRIPES[r]
                cur = pending[r]
                slot_by_s = {s: slot for (_, slot, s) in cur}
                if t < 2:
                    fwd = {order[t + 1] ^ s2 for s2 in _span(order[t + 2:])}
                    for rdma, slot, s in cur:
                        if s in fwd:
                            rdma.wait_recv()
                    pending[r] = issue_rs(r, t + 1, slot_by_s)
                    for rdma, slot, s in cur:
                        if s not in fwd:
                            rdma.wait_recv()
                            rows = pl.ds((my ^ s) * CH, CH)
                            part_ref[rows, pl.ds(c0, w)] = (
                                part_ref[rows, pl.ds(c0, w)]
                                + rbufs[r][slot, :, :].astype(jnp.float32)
                            )
                else:
                    rdma, slot, s = cur[0]
                    rdma.wait_recv()
                    rows = pl.ds(my * CH, CH)
                    z = (
                        part_ref[rows, pl.ds(c0, w)]
                        + rbufs[r][slot, :, :].astype(jnp.float32)
                    )
                    act = z * jax.nn.sigmoid(z)
                    out_ref[rows, pl.ds(c0, w)] = act
                    gbufs[r][pl.ds(my, 1), :, :] = act.astype(jnp.bfloat16)[
                        None, :, :
                    ]
                    pending[r] = issue_ag(r, 0)

        for t in range(3):
            for r in range(3):
                c0, w = STRIPES[r]
                arrived = []
                for rdma, c in pending[r]:
                    rdma.wait_recv()
                    arrived.append(c)
                pending[r] = issue_ag(r, t + 1) if t < 2 else []
                for c in arrived:
                    out_ref[pl.ds(c * CH, CH), pl.ds(c0, w)] = gbufs[r][
                        c, :, :
                    ].astype(jnp.float32)

        for rdma in all_rdmas:
            rdma.wait_send()

    return pl.pallas_call(
        body,
        out_shape=jax.ShapeDtypeStruct((m, n), jnp.float32),
        in_specs=[
            pl.BlockSpec(memory_space=pltpu.VMEM),
            pl.BlockSpec(memory_space=pltpu.VMEM),
        ],
        out_specs=pl.BlockSpec(memory_space=pltpu.VMEM),
        scratch_shapes=[
            pltpu.VMEM((m, n), jnp.float32),
            pltpu.VMEM((7, CH, 384), jnp.bfloat16),
            pltpu.VMEM((7, CH, 384), jnp.bfloat16),
            pltpu.VMEM((7, CH, 256), jnp.bfloat16),
            pltpu.VMEM((7, CH, 384), jnp.bfloat16),
            pltpu.VMEM((7, CH, 384), jnp.bfloat16),
            pltpu.VMEM((7, CH, 256), jnp.bfloat16),
            pltpu.VMEM((N_DEV, CH, 384), jnp.bfloat16),
            pltpu.VMEM((N_DEV, CH, 384), jnp.bfloat16),
            pltpu.VMEM((N_DEV, CH, 256), jnp.bfloat16),
            pltpu.SemaphoreType.DMA((3, 7)),
            pltpu.SemaphoreType.DMA((3, 7)),
            pltpu.SemaphoreType.DMA((3, 7)),
            pltpu.SemaphoreType.DMA((3, 7)),
        ],
        compiler_params=pltpu.CompilerParams(collective_id=0),
    )(A, B)


# device time: 32162 ns/iter; 3.5462x vs baseline; 1.1108x over previous
import jax
import jax.numpy as jnp
from jax import lax
from jax.experimental import pallas as pl
from jax.experimental.pallas import tpu as pltpu

N_DEV = 8
CH = 128

X, Y, Z = 1, 3, 4
ORDERS = ((X, Y, Z), (Y, Z, X), (Z, X, Y))
STRIPES = ((0, 384), (384, 384), (768, 256))


def _span(masks):
    out = [0]
    for m in masks:
        out = out + [v ^ m for v in out]
    return out


def kernel(A, B):
    m, _ = A.shape
    _, n = B.shape

    def body(a_ref, b_ref, out_ref, part_ref,
             rbuf0, rbuf1, rbuf2, sbuf0, sbuf1, sbuf2,
             gbuf0, gbuf1, gbuf2,
             rs_send, rs_recv, ag_send, ag_recv):
        rbufs = (rbuf0, rbuf1, rbuf2)
        sbufs = (sbuf0, sbuf1, sbuf2)
        gbufs = (gbuf0, gbuf1, gbuf2)
        my = lax.axis_index("i")
        all_rdmas = []

        barrier_sem = pltpu.get_barrier_semaphore()
        for mask in (X, Y, Z):
            pl.semaphore_signal(
                barrier_sem, inc=1,
                device_id=(my ^ mask,), device_id_type=pl.DeviceIdType.MESH,
            )

        def issue_rs(r, t, prev_slot_by_s, prestaged=False):
            order = ORDERS[r]
            mask = order[t]
            rest = _span(order[t + 1:])
            c0, w = STRIPES[r]
            slot0 = 8 - 2 * len(rest)
            out = []
            for k, s in enumerate(rest):
                slot = slot0 + k
                c_send = my ^ mask ^ s
                if not prestaged:
                    rows = pl.ds(c_send * CH, CH)
                    val = part_ref[rows, pl.ds(c0, w)]
                    if t > 0:
                        pslot = prev_slot_by_s[mask ^ s]
                        val = val + rbufs[r][pslot, :, :].astype(jnp.float32)
                    sbufs[r][slot, :, :] = val.astype(jnp.bfloat16)
                rdma = pltpu.make_async_remote_copy(
                    src_ref=sbufs[r].at[slot],
                    dst_ref=rbufs[r].at[slot],
                    send_sem=rs_send.at[r, slot],
                    recv_sem=rs_recv.at[r, slot],
                    device_id=(my ^ mask,),
                    device_id_type=pl.DeviceIdType.MESH,
                )
                rdma.start()
                all_rdmas.append(rdma)
                out.append((rdma, slot, s))
            return out

        def ag_plan(r):
            ag_order = ORDERS[r][::-1]
            sends = {}
            recv_label = {}
            for t in range(3):
                have = _span(ag_order[:t])
                slot0 = len(have) - 1
                for k, s in enumerate(have):
                    sends.setdefault(s, []).append(
                        (slot0 + k, ag_order[t])
                    )
                    recv_label[slot0 + k] = ag_order[t] ^ s
            return sends, recv_label

        ag_pend = {r: {} for r in range(3)}

        def issue_ag_chunk(r, lam):
            sends, _ = ag_plan(r)
            c_send = my ^ lam
            for slot, mask in sends.get(lam, []):
                rdma = pltpu.make_async_remote_copy(
                    src_ref=gbufs[r].at[c_send],
                    dst_ref=gbufs[r].at[c_send],
                    send_sem=ag_send.at[r, slot],
                    recv_sem=ag_recv.at[r, slot],
                    device_id=(my ^ mask,),
                    device_id_type=pl.DeviceIdType.MESH,
                )
                rdma.start()
                all_rdmas.append(rdma)
                ag_pend[r][slot] = rdma

        for r in range(3):
            order = ORDERS[r]
            c0, w = STRIPES[r]
            part_ref[:, pl.ds(c0, w)] = jnp.dot(
                a_ref[...], b_ref[:, pl.ds(c0, w)],
                preferred_element_type=jnp.float32,
            )
            for k, s in enumerate(_span(order[1:])):
                c_send = my ^ order[0] ^ s
                sbufs[r][k, :, :] = part_ref[
                    pl.ds(c_send * CH, CH), pl.ds(c0, w)
                ].astype(jnp.bfloat16)
        pl.semaphore_wait(barrier_sem, 3)
        pending = {r: issue_rs(r, 0, None, prestaged=True) for r in range(3)}

        for t in range(3):
            for r in range(3):
                order = ORDERS[r]
                c0, w = STRIPES[r]
                cur = pending[r]
                slot_by_s = {s: slot for (_, slot, s) in cur}
                if t < 2:
                    fwd = {order[t + 1] ^ s2 for s2 in _span(order[t + 2:])}
                    for rdma, slot, s in cur:
                        if s in fwd:
                            rdma.wait_recv()
                    pending[r] = issue_rs(r, t + 1, slot_by_s)
                    for rdma, slot, s in cur:
                        if s not in fwd:
                            rdma.wait_recv()
                            rows = pl.ds((my ^ s) * CH, CH)
                            part_ref[rows, pl.ds(c0, w)] = (
                                part_ref[rows, pl.ds(c0, w)]
                                + rbufs[r][slot, :, :].astype(jnp.float32)
                            )
                else:
                    rdma, slot, s = cur[0]
                    rdma.wait_recv()
                    rows = pl.ds(my * CH, CH)
                    z = (
                        part_ref[rows, pl.ds(c0, w)]
                        + rbufs[r][slot, :, :].astype(jnp.float32)
                    )
                    act = z * jax.nn.sigmoid(z)
                    out_ref[rows, pl.ds(c0, w)] = act
                    gbufs[r][pl.ds(my, 1), :, :] = act.astype(jnp.bfloat16)[
                        None, :, :
                    ]
                    issue_ag_chunk(r, 0)

        for slot in range(7):
            for r in range(3):
                c0, w = STRIPES[r]
                _, recv_label = ag_plan(r)
                lam = recv_label[slot]
                ag_pend[r][slot].wait_recv()
                issue_ag_chunk(r, lam)
                c = my ^ lam
                out_ref[pl.ds(c * CH, CH), pl.ds(c0, w)] = gbufs[r][
                    c, :, :
                ].astype(jnp.float32)

        for rdma in all_rdmas:
            rdma.wait_send()

    return pl.pallas_call(
        body,
        out_shape=jax.ShapeDtypeStruct((m, n), jnp.float32),
        in_specs=[
            pl.BlockSpec(memory_space=pltpu.VMEM),
            pl.BlockSpec(memory_space=pltpu.VMEM),
        ],
        out_specs=pl.BlockSpec(memory_space=pltpu.VMEM),
        scratch_shapes=[
            pltpu.VMEM((m, n), jnp.float32),
            pltpu.VMEM((7, CH, 384), jnp.bfloat16),
            pltpu.VMEM((7, CH, 384), jnp.bfloat16),
            pltpu.VMEM((7, CH, 256), jnp.bfloat16),
            pltpu.VMEM((7, CH, 384), jnp.bfloat16),
            pltpu.VMEM((7, CH, 384), jnp.bfloat16),
            pltpu.VMEM((7, CH, 256), jnp.bfloat16),
            pltpu.VMEM((N_DEV, CH, 384), jnp.bfloat16),
            pltpu.VMEM((N_DEV, CH, 384), jnp.bfloat16),
            pltpu.VMEM((N_DEV, CH, 256), jnp.bfloat16),
            pltpu.SemaphoreType.DMA((3, 7)),
            pltpu.SemaphoreType.DMA((3, 7)),
            pltpu.SemaphoreType.DMA((3, 7)),
            pltpu.SemaphoreType.DMA((3, 7)),
        ],
        compiler_params=pltpu.CompilerParams(collective_id=0),
    )(A, B)
